# Initial kernel scaffold; baseline (speedup 1.0000x reference)
import jax
import jax.numpy as jnp
from jax.experimental import pallas as pl


def kernel(hidden_states, router, gate_proj, up_proj, down_proj):
    raise NotImplementedError("write your pallas kernel here")


# speedup vs baseline: 2.2605x; 2.2605x over previous
import functools

import jax
import jax.numpy as jnp
from jax.experimental import pallas as pl
from jax.experimental.pallas import tpu as pltpu

B, S, D = 16, 32, 1024
E, TOPK, DFF = 8, 2, 1024
T = B * S


def _moe_dense_body(x_ref, r_ref, g_ref, u_ref, d_ref, o_ref, w_ref):
    e = pl.program_id(0)

    @pl.when(e == 0)
    def _route():
        x = x_ref[...]
        logits = jax.lax.dot_general(
            x, r_ref[...], (((1,), (1,)), ((), ())),
            preferred_element_type=jnp.float32)
        idx = jax.lax.broadcasted_iota(jnp.int32, (T, E), 1)
        m0 = jnp.max(logits, axis=1, keepdims=True)
        i0 = jnp.min(jnp.where(logits == m0, idx, E), axis=1, keepdims=True)
        masked = jnp.where(idx == i0, -jnp.inf, logits)
        m1 = jnp.max(masked, axis=1, keepdims=True)
        i1 = jnp.min(jnp.where(masked == m1, idx, E), axis=1, keepdims=True)
        e1 = jnp.exp(m1 - m0)
        denom = 1.0 + e1
        w0 = 1.0 / denom
        w1 = e1 / denom
        wsum = w0 + w1
        w0 = w0 / wsum
        w1 = w1 / wsum
        w_ref[...] = jnp.where(idx == i0, w0, 0.0) + jnp.where(idx == i1, w1, 0.0)

    x = x_ref[...]
    gate = jax.lax.dot_general(
        x, g_ref[0], (((1,), (1,)), ((), ())), preferred_element_type=jnp.float32)
    up = jax.lax.dot_general(
        x, u_ref[0], (((1,), (1,)), ((), ())), preferred_element_type=jnp.float32)
    inter = gate * jax.lax.logistic(gate) * up
    eo = jax.lax.dot_general(
        inter, d_ref[0], (((1,), (1,)), ((), ())), preferred_element_type=jnp.float32)
    idx = jax.lax.broadcasted_iota(jnp.int32, (T, E), 1)
    wcol = jnp.sum(w_ref[...] * jnp.where(idx == e, 1.0, 0.0), axis=1, keepdims=True)
    contrib = eo * wcol

    @pl.when(e == 0)
    def _init():
        o_ref[...] = contrib

    @pl.when(e > 0)
    def _acc():
        o_ref[...] += contrib


@jax.jit
def _moe_dense(x, router, gate_proj, up_proj, down_proj):
    out = pl.pallas_call(
        _moe_dense_body,
        grid=(E,),
        in_specs=[
            pl.BlockSpec((T, D), lambda e: (0, 0)),
            pl.BlockSpec((E, D), lambda e: (0, 0)),
            pl.BlockSpec((1, DFF, D), lambda e: (e, 0, 0)),
            pl.BlockSpec((1, DFF, D), lambda e: (e, 0, 0)),
            pl.BlockSpec((1, D, DFF), lambda e: (e, 0, 0)),
        ],
        out_specs=pl.BlockSpec((T, D), lambda e: (0, 0)),
        out_shape=jax.ShapeDtypeStruct((T, D), jnp.float32),
        scratch_shapes=[pltpu.VMEM((T, E), jnp.float32)],
    )(x, router, gate_proj, up_proj, down_proj)
    return out


def kernel(hidden_states, router, gate_proj, up_proj, down_proj):
    b, s, d = hidden_states.shape
    x = hidden_states.reshape(-1, d)
    out = _moe_dense(x, router, gate_proj, up_proj, down_proj)
    return out.reshape(b, s, d)


# speedup vs baseline: 2.2941x; 1.0148x over previous
import jax
import jax.numpy as jnp
from jax.experimental import pallas as pl
from jax.experimental.pallas import tpu as pltpu

B, S, D = 16, 32, 1024
E, TOPK, DFF = 8, 2, 1024
T = B * S


def _moe_dense_body(x_ref, r_ref, g_ref, u_ref, d_ref, o_ref, w_ref, xb_ref):
    e = pl.program_id(0)

    @pl.when(e == 0)
    def _route():
        x = x_ref[...]
        xb_ref[...] = x.astype(jnp.bfloat16)
        logits = jax.lax.dot_general(
            x, r_ref[...], (((1,), (1,)), ((), ())),
            preferred_element_type=jnp.float32)
        idx = jax.lax.broadcasted_iota(jnp.int32, (T, E), 1)
        m0 = jnp.max(logits, axis=1, keepdims=True)
        i0 = jnp.min(jnp.where(logits == m0, idx, E), axis=1, keepdims=True)
        masked = jnp.where(idx == i0, -jnp.inf, logits)
        m1 = jnp.max(masked, axis=1, keepdims=True)
        i1 = jnp.min(jnp.where(masked == m1, idx, E), axis=1, keepdims=True)
        e1 = jnp.exp(m1 - m0)
        denom = 1.0 + e1
        w0 = 1.0 / denom
        w1 = e1 / denom
        wsum = w0 + w1
        w0 = w0 / wsum
        w1 = w1 / wsum
        w_ref[...] = jnp.where(idx == i0, w0, 0.0) + jnp.where(idx == i1, w1, 0.0)

    xb = xb_ref[...]
    gate = jax.lax.dot_general(
        xb, g_ref[0].astype(jnp.bfloat16), (((1,), (1,)), ((), ())),
        preferred_element_type=jnp.float32)
    up = jax.lax.dot_general(
        xb, u_ref[0].astype(jnp.bfloat16), (((1,), (1,)), ((), ())),
        preferred_element_type=jnp.float32)
    inter = gate * jax.lax.logistic(gate) * up
    eo = jax.lax.dot_general(
        inter.astype(jnp.bfloat16), d_ref[0].astype(jnp.bfloat16),
        (((1,), (1,)), ((), ())), preferred_element_type=jnp.float32)
    idx = jax.lax.broadcasted_iota(jnp.int32, (T, E), 1)
    wcol = jnp.sum(w_ref[...] * jnp.where(idx == e, 1.0, 0.0), axis=1, keepdims=True)
    contrib = eo * wcol

    @pl.when(e == 0)
    def _init():
        o_ref[...] = contrib

    @pl.when(e > 0)
    def _acc():
        o_ref[...] += contrib


@jax.jit
def _moe_dense(x, router, gate_proj, up_proj, down_proj):
    out = pl.pallas_call(
        _moe_dense_body,
        grid=(E,),
        in_specs=[
            pl.BlockSpec((T, D), lambda e: (0, 0)),
            pl.BlockSpec((E, D), lambda e: (0, 0)),
            pl.BlockSpec((1, DFF, D), lambda e: (e, 0, 0)),
            pl.BlockSpec((1, DFF, D), lambda e: (e, 0, 0)),
            pl.BlockSpec((1, D, DFF), lambda e: (e, 0, 0)),
        ],
        out_specs=pl.BlockSpec((T, D), lambda e: (0, 0)),
        out_shape=jax.ShapeDtypeStruct((T, D), jnp.float32),
        scratch_shapes=[
            pltpu.VMEM((T, E), jnp.float32),
            pltpu.VMEM((T, D), jnp.bfloat16),
        ],
    )(x, router, gate_proj, up_proj, down_proj)
    return out


def kernel(hidden_states, router, gate_proj, up_proj, down_proj):
    b, s, d = hidden_states.shape
    x = hidden_states.reshape(-1, d)
    out = _moe_dense(x, router, gate_proj, up_proj, down_proj)
    return out.reshape(b, s, d)
